# baseline (speedup 1.0000x reference)
import jax
import jax.numpy as jnp
from jax.experimental import pallas as pl
from jax.experimental.pallas import tpu as pltpu

_RBLK = 64


def _body(sid_ref, pts_ref, feat_ref, out_ref):
    c = pts_ref.shape[1]
    rows, n = out_ref.shape[1], out_ref.shape[2]
    out_ref[0] = jnp.broadcast_to(feat_ref[0], (rows, n))

    @pl.when(pl.program_id(1) == 0)
    def _():
        out_ref[0, :c, :] = pts_ref[0]


def kernel(points, subject_garment_id, features):
    b, c, n = points.shape
    s, l = features.shape
    rows = c + l
    n_rblk = -(-rows // _RBLK)
    rows_pad = n_rblk * _RBLK
    grid = (b, n_rblk)
    feats_pad = jnp.concatenate(
        [jnp.zeros((s, c), jnp.float32), features,
         jnp.zeros((s, rows_pad - rows), jnp.float32)], axis=1
    ).reshape(s, rows_pad, 1)

    return pl.pallas_call(
        _body,
        grid_spec=pltpu.PrefetchScalarGridSpec(
            num_scalar_prefetch=1,
            grid=grid,
            in_specs=[
                pl.BlockSpec((1, c, n), lambda bi, ri, sid: (bi, 0, 0)),
                pl.BlockSpec((1, _RBLK, 1), lambda bi, ri, sid: (sid[bi], ri, 0)),
            ],
            out_specs=pl.BlockSpec((1, _RBLK, n), lambda bi, ri, sid: (bi, ri, 0)),
        ),
        out_shape=jax.ShapeDtypeStruct((b, rows, n), jnp.float32),
    )(subject_garment_id, points, feats_pad)
